# Initial kernel scaffold; baseline (speedup 1.0000x reference)
import jax
import jax.numpy as jnp
from jax.experimental import pallas as pl


def kernel(x):
    raise NotImplementedError("write your pallas kernel here")


# speedup vs baseline: 2.8596x; 2.8596x over previous
import jax
import jax.numpy as jnp
from jax import lax
from jax.experimental import pallas as pl
from jax.experimental.pallas import tpu as pltpu

_K = 9
_DIL = 2
_KSEL = (_K - 1) * _DIL + 1


def _knn_sel_kernel(q_ref, k_ref, khalf_ref, idx_ref):
    q = q_ref[0]
    kn = k_ref[0]
    tq = q.shape[1]
    nk = kn.shape[1]

    gram = lax.dot_general(q, kn, (((0,), (0,)), ((), ())),
                           preferred_element_type=jnp.float32)
    d = khalf_ref[0] - gram

    gidx = lax.broadcasted_iota(jnp.int32, (1, nk), 1)
    big = jnp.int32(2 ** 30)
    out_iota = lax.broadcasted_iota(jnp.int32, (1, _K), 1)
    acc = jnp.zeros((tq, _K), jnp.int32)

    for r in range(_KSEL):
        dmin = jnp.min(d, axis=-1, keepdims=True)
        cand = jnp.where(d <= dmin, gidx, big)
        sel = jnp.min(cand, axis=-1, keepdims=True)
        if r % _DIL == 0:
            acc = jnp.where(out_iota == (r // _DIL), sel, acc)
        if r + 1 < _KSEL:
            d = jnp.where(cand == sel, jnp.float32(jnp.inf), d)

    idx_ref[0] = acc.T


def _l2_normalize(x_bcn, eps=1e-12):
    ssq = jnp.sum(x_bcn * x_bcn, axis=1, keepdims=True)
    return x_bcn * lax.rsqrt(jnp.maximum(ssq, eps * eps))


def kernel(x):
    B, C, N, W = x.shape
    assert W == 1
    xn = _l2_normalize(x[..., 0].astype(jnp.float32))
    khalf = 0.5 * jnp.sum(xn * xn, axis=1, keepdims=True)

    tq = 256 if N % 256 == 0 else 128
    assert N % tq == 0 and _KSEL <= N

    idx = pl.pallas_call(
        _knn_sel_kernel,
        out_shape=jax.ShapeDtypeStruct((B, _K, N), jnp.int32),
        grid=(B, N // tq),
        in_specs=[
            pl.BlockSpec((1, C, tq), lambda b, t: (b, 0, t)),
            pl.BlockSpec((1, C, N), lambda b, t: (b, 0, 0)),
            pl.BlockSpec((1, 1, N), lambda b, t: (b, 0, 0)),
        ],
        out_specs=pl.BlockSpec((1, _K, tq), lambda b, t: (b, 0, t)),
        compiler_params=pltpu.CompilerParams(
            dimension_semantics=("parallel", "parallel"),
            vmem_limit_bytes=64 * 1024 * 1024,
        ),
    )(xn, xn, khalf)

    nn_idx = jnp.transpose(idx, (0, 2, 1))
    center = jnp.broadcast_to(
        jnp.arange(N, dtype=jnp.int32)[None, :, None], (B, N, _K))
    return jnp.stack([nn_idx, center], axis=0)


# speedup vs baseline: 3.7436x; 1.3091x over previous
import jax
import jax.numpy as jnp
from jax import lax
from jax.experimental import pallas as pl
from jax.experimental.pallas import tpu as pltpu

_K = 9
_DIL = 2
_KSEL = (_K - 1) * _DIL + 1


def _knn_sel_kernel(q_ref, k_ref, khalf_ref, idx_ref):
    q = q_ref[0]
    kn = k_ref[0]
    tq = q.shape[1]
    nk = kn.shape[1]

    gram = lax.dot_general(q, kn, (((0,), (0,)), ((), ())),
                           preferred_element_type=jnp.float32)
    d = khalf_ref[0] - gram

    gidx = lax.broadcasted_iota(jnp.int32, (1, nk), 1).astype(jnp.float32)
    big = jnp.float32(2 ** 30)
    out_iota = lax.broadcasted_iota(jnp.int32, (1, _K), 1)
    acc = jnp.zeros((tq, _K), jnp.float32)

    for r in range(_KSEL):
        dmin = jnp.min(d, axis=-1, keepdims=True)
        cand = jnp.where(d <= dmin, gidx, big)
        sel = jnp.min(cand, axis=-1, keepdims=True)
        if r % _DIL == 0:
            acc = jnp.where(out_iota == (r // _DIL), sel, acc)
        if r + 1 < _KSEL:
            d = jnp.where(cand == sel, jnp.float32(jnp.inf), d)

    idx_ref[0] = acc.astype(jnp.int32).T


def _l2_normalize(x_bcn, eps=1e-12):
    ssq = jnp.sum(x_bcn * x_bcn, axis=1, keepdims=True)
    return x_bcn * lax.rsqrt(jnp.maximum(ssq, eps * eps))


def kernel(x):
    B, C, N, W = x.shape
    assert W == 1
    xn = _l2_normalize(x[..., 0].astype(jnp.float32))
    khalf = 0.5 * jnp.sum(xn * xn, axis=1, keepdims=True)

    tq = 256 if N % 256 == 0 else 128
    assert N % tq == 0 and _KSEL <= N

    idx = pl.pallas_call(
        _knn_sel_kernel,
        out_shape=jax.ShapeDtypeStruct((B, _K, N), jnp.int32),
        grid=(B, N // tq),
        in_specs=[
            pl.BlockSpec((1, C, tq), lambda b, t: (b, 0, t)),
            pl.BlockSpec((1, C, N), lambda b, t: (b, 0, 0)),
            pl.BlockSpec((1, 1, N), lambda b, t: (b, 0, 0)),
        ],
        out_specs=pl.BlockSpec((1, _K, tq), lambda b, t: (b, 0, t)),
        compiler_params=pltpu.CompilerParams(
            dimension_semantics=("parallel", "parallel"),
            vmem_limit_bytes=64 * 1024 * 1024,
        ),
    )(xn, xn, khalf)

    nn_idx = jnp.transpose(idx, (0, 2, 1))
    center = jnp.broadcast_to(
        jnp.arange(N, dtype=jnp.int32)[None, :, None], (B, N, _K))
    return jnp.stack([nn_idx, center], axis=0)
